# Initial kernel scaffold; baseline (speedup 1.0000x reference)
import jax
import jax.numpy as jnp
from jax.experimental import pallas as pl


def kernel(x, edge_index, W_pre, b_pre, W_conv, b_conv):
    raise NotImplementedError("write your pallas kernel here")


# speedup vs baseline: 17.6898x; 17.6898x over previous
import functools

import jax
import jax.numpy as jnp
from jax import lax
from jax.experimental import pallas as pl
from jax.experimental.pallas import tpu as pltpu
from jax.experimental.pallas import tpu_sc as plsc

N_PAD = 10240
D = 128
NC, NS = 2, 16
NW = NC * NS
CHUNK = 80
ROWS_PT = N_PAD // NS


def _sc_mesh():
    return plsc.VectorSubcoreMesh(
        core_axis_name="c", subcore_axis_name="s", num_cores=NC, num_subcores=NS
    )


def _sc_degree(dst, n_edges):
    edges_pt = n_edges // NW
    n_chunks = edges_pt // CHUNK

    @functools.partial(
        pl.kernel,
        mesh=_sc_mesh(),
        out_type=jax.ShapeDtypeStruct((NC, N_PAD), jnp.float32),
        scratch_types=[
            pltpu.VMEM((CHUNK,), jnp.int32),
            pltpu.VMEM((CHUNK,), jnp.float32),
            pltpu.VMEM((ROWS_PT,), jnp.float32),
            pltpu.VMEM_SHARED((N_PAD,), jnp.float32),
        ],
    )
    def k(dst_hbm, out_hbm, idx_v, ones_v, zeros_v, acc_sh):
        c = lax.axis_index("c")
        s = lax.axis_index("s")
        for j in range(CHUNK // 16):
            ones_v[pl.ds(j * 16, 16)] = jnp.full((16,), 1.0, jnp.float32)
        for j in range(ROWS_PT // 16):
            zeros_v[pl.ds(j * 16, 16)] = jnp.zeros((16,), jnp.float32)
        pltpu.sync_copy(zeros_v, acc_sh.at[pl.ds(s * ROWS_PT, ROWS_PT)])
        plsc.subcore_barrier()

        base0 = (c * NS + s) * edges_pt

        def body(i, carry):
            base = base0 + i * CHUNK
            pltpu.sync_copy(dst_hbm.at[pl.ds(base, CHUNK)], idx_v)
            pltpu.sync_copy(ones_v, acc_sh.at[idx_v], add=True)
            return carry

        lax.fori_loop(0, n_chunks, body, 0)
        plsc.subcore_barrier()
        pltpu.sync_copy(
            acc_sh.at[pl.ds(s * ROWS_PT, ROWS_PT)],
            out_hbm.at[c, pl.ds(s * ROWS_PT, ROWS_PT)],
        )

    return k(dst)


def _sc_scatter(g, src, dst, n_edges):
    edges_pt = n_edges // NW
    n_chunks = edges_pt // CHUNK

    @functools.partial(
        pl.kernel,
        mesh=_sc_mesh(),
        out_type=jax.ShapeDtypeStruct((NC, N_PAD, D), jnp.float32),
        scratch_types=[
            pltpu.VMEM((CHUNK,), jnp.int32),
            pltpu.VMEM((CHUNK,), jnp.int32),
            pltpu.VMEM((CHUNK, D), jnp.float32),
            pltpu.VMEM((CHUNK, D), jnp.float32),
            pltpu.VMEM_SHARED((N_PAD, D), jnp.float32),
            pltpu.SemaphoreType.DMA,
        ],
    )
    def k(g_hbm, src_hbm, dst_hbm, out_hbm, si, di, rows, zbuf, acc_sh, sem):
        c = lax.axis_index("c")
        s = lax.axis_index("s")

        def zrow(i, carry):
            for j in range(D // 16):
                zbuf[i, pl.ds(j * 16, 16)] = jnp.zeros((16,), jnp.float32)
            return carry

        lax.fori_loop(0, CHUNK, zrow, 0)
        for t in range(ROWS_PT // CHUNK):
            pltpu.sync_copy(zbuf, acc_sh.at[pl.ds(s * ROWS_PT + t * CHUNK, CHUNK)])
        plsc.subcore_barrier()

        base0 = (c * NS + s) * edges_pt

        def body(i, carry):
            base = base0 + i * CHUNK
            pltpu.sync_copy(src_hbm.at[pl.ds(base, CHUNK)], si)
            pltpu.sync_copy(dst_hbm.at[pl.ds(base, CHUNK)], di)
            pltpu.async_copy(g_hbm.at[si], rows, sem).wait()
            pltpu.sync_copy(rows, acc_sh.at[di], add=True)
            return carry

        lax.fori_loop(0, n_chunks, body, 0)
        plsc.subcore_barrier()
        pltpu.sync_copy(
            acc_sh.at[pl.ds(s * ROWS_PT, ROWS_PT)],
            out_hbm.at[c, pl.ds(s * ROWS_PT, ROWS_PT)],
        )

    return k(g, src, dst)


def _tc_prep(x_pad, W_pre, b_pre2, W_conv, deg_b):
    R = 1280
    grid = N_PAD // R

    def body(x_ref, wp_ref, bp_ref, wc_ref, degb_ref, ori_ref, g_ref):
        ori = (
            jnp.dot(x_ref[...], wp_ref[...], preferred_element_type=jnp.float32)
            + bp_ref[...]
        )
        ori_ref[...] = ori
        h2 = jnp.dot(ori, wc_ref[...], preferred_element_type=jnp.float32)
        g_ref[...] = lax.rsqrt(degb_ref[...]) * h2

    return pl.pallas_call(
        body,
        grid=(grid,),
        in_specs=[
            pl.BlockSpec((R, D), lambda i: (i, 0)),
            pl.BlockSpec((D, D), lambda i: (0, 0)),
            pl.BlockSpec((1, D), lambda i: (0, 0)),
            pl.BlockSpec((D, D), lambda i: (0, 0)),
            pl.BlockSpec((R, D), lambda i: (i, 0)),
        ],
        out_specs=[
            pl.BlockSpec((R, D), lambda i: (i, 0)),
            pl.BlockSpec((R, D), lambda i: (i, 0)),
        ],
        out_shape=[
            jax.ShapeDtypeStruct((N_PAD, D), jnp.float32),
            jax.ShapeDtypeStruct((N_PAD, D), jnp.float32),
        ],
    )(x_pad, W_pre, b_pre2, W_conv, deg_b)


def _tc_finish(P, g, deg_b, b_conv2):
    R = 1280
    grid = N_PAD // R

    def body(p_ref, g_ref, degb_ref, bc_ref, out_ref):
        tot = p_ref[0] + p_ref[1] + g_ref[...]
        out_ref[...] = jnp.maximum(
            lax.rsqrt(degb_ref[...]) * tot + bc_ref[...], 0.0
        )

    return pl.pallas_call(
        body,
        grid=(grid,),
        in_specs=[
            pl.BlockSpec((NC, R, D), lambda i: (0, i, 0)),
            pl.BlockSpec((R, D), lambda i: (i, 0)),
            pl.BlockSpec((R, D), lambda i: (i, 0)),
            pl.BlockSpec((1, D), lambda i: (0, 0)),
        ],
        out_specs=pl.BlockSpec((R, D), lambda i: (i, 0)),
        out_shape=jax.ShapeDtypeStruct((N_PAD, D), jnp.float32),
    )(P, g, deg_b, b_conv2)


def kernel(x, edge_index, W_pre, b_pre, W_conv, b_conv):
    n = x.shape[0]
    n_edges = edge_index.shape[1]
    src = edge_index[0]
    dst = edge_index[1]
    x_pad = jnp.pad(x, ((0, N_PAD - n), (0, 0)))

    deg_parts = _sc_degree(dst, n_edges)
    deg = deg_parts[0] + deg_parts[1] + 1.0
    deg_b = jnp.broadcast_to(deg[:, None], (N_PAD, D))

    ori_pad, g_pad = _tc_prep(x_pad, W_pre, b_pre[None, :], W_conv, deg_b)
    P = _sc_scatter(g_pad, src, dst, n_edges)
    h_pad = _tc_finish(P, g_pad, deg_b, b_conv[None, :])
    return h_pad[:n], ori_pad[:n]


# speedup vs baseline: 33.9970x; 1.9218x over previous
import functools

import jax
import jax.numpy as jnp
from jax import lax
from jax.experimental import pallas as pl
from jax.experimental.pallas import tpu as pltpu
from jax.experimental.pallas import tpu_sc as plsc

N_PAD = 10240
D = 128
NC, NS = 2, 16
NW = NC * NS
CHUNK = 80
NCHUNK = 125
ROWS_PT = N_PAD // NS
ZROWS = 16


def _sc_mesh():
    return plsc.VectorSubcoreMesh(
        core_axis_name="c", subcore_axis_name="s", num_cores=NC, num_subcores=NS
    )


def _sc_degree(dst3):
    @functools.partial(
        pl.kernel,
        mesh=_sc_mesh(),
        out_type=jax.ShapeDtypeStruct((NC, N_PAD), jnp.float32),
        scratch_types=[
            pltpu.VMEM((NCHUNK, CHUNK), jnp.int32),
            pltpu.VMEM((128,), jnp.float32),
            pltpu.VMEM((ROWS_PT,), jnp.float32),
            pltpu.VMEM_SHARED((N_PAD,), jnp.float32),
            pltpu.SemaphoreType.DMA,
        ],
    )
    def k(dst_hbm, out_hbm, dst_v, ones_v, zeros_v, acc_sh, sem):
        c = lax.axis_index("c")
        s = lax.axis_index("s")
        wid = c * NS + s
        pltpu.sync_copy(dst_hbm.at[wid], dst_v)
        for j in range(128 // 16):
            ones_v[pl.ds(j * 16, 16)] = jnp.full((16,), 1.0, jnp.float32)
        for j in range(ROWS_PT // 16):
            zeros_v[pl.ds(j * 16, 16)] = jnp.zeros((16,), jnp.float32)
        pltpu.sync_copy(zeros_v, acc_sh.at[pl.ds(s * ROWS_PT, ROWS_PT)])
        plsc.subcore_barrier()

        ones_c = ones_v.at[pl.ds(0, CHUNK)]

        def body(i, carry):
            for j in range(5):
                pltpu.async_copy(
                    ones_c, acc_sh.at[dst_v.at[i * 5 + j]], sem, add=True
                )
            for j in range(5):
                pltpu.make_async_copy(
                    ones_c, acc_sh.at[dst_v.at[i * 5 + j]], sem
                ).wait()
            return carry

        lax.fori_loop(0, NCHUNK // 5, body, 0)
        plsc.subcore_barrier()
        pltpu.sync_copy(
            acc_sh.at[pl.ds(s * ROWS_PT, ROWS_PT)],
            out_hbm.at[c, pl.ds(s * ROWS_PT, ROWS_PT)],
        )

    return k(dst3)


def _sc_scatter(g, src3, dst3):
    @functools.partial(
        pl.kernel,
        mesh=_sc_mesh(),
        out_type=jax.ShapeDtypeStruct((NC, N_PAD, D), jnp.float32),
        scratch_types=[
            pltpu.VMEM((NCHUNK, CHUNK), jnp.int32),
            pltpu.VMEM((CHUNK,), jnp.int32),
            pltpu.VMEM((CHUNK,), jnp.int32),
            pltpu.VMEM((CHUNK, D), jnp.float32),
            pltpu.VMEM((CHUNK, D), jnp.float32),
            pltpu.VMEM((ZROWS, D), jnp.float32),
            pltpu.VMEM_SHARED((N_PAD, D), jnp.float32),
            pltpu.SemaphoreType.DMA,
            pltpu.SemaphoreType.DMA,
            pltpu.SemaphoreType.DMA,
            pltpu.SemaphoreType.DMA,
            pltpu.SemaphoreType.DMA,
            pltpu.SemaphoreType.DMA,
        ],
    )
    def k(g_hbm, src_hbm, dst_hbm, out_hbm,
          src_v, dst_a, dst_b, rows_a, rows_b, zbuf, acc_sh,
          gs_a, gs_b, ss_a, ss_b, ds_a, ds_b):
        c = lax.axis_index("c")
        s = lax.axis_index("s")
        wid = c * NS + s
        pltpu.sync_copy(src_hbm.at[wid], src_v)

        def zrow(i, carry):
            for j in range(D // 16):
                zbuf[i, pl.ds(j * 16, 16)] = jnp.zeros((16,), jnp.float32)
            return carry

        lax.fori_loop(0, ZROWS, zrow, 0)
        for t in range(ROWS_PT // ZROWS):
            pltpu.sync_copy(zbuf, acc_sh.at[pl.ds(s * ROWS_PT + t * ZROWS, ZROWS)])
        plsc.subcore_barrier()

        pltpu.async_copy(g_hbm.at[src_v.at[0]], rows_a, gs_a)
        pltpu.async_copy(dst_hbm.at[wid * NCHUNK + 0], dst_a, ds_a)
        pltpu.async_copy(g_hbm.at[src_v.at[1]], rows_b, gs_b)
        pltpu.async_copy(dst_hbm.at[wid * NCHUNK + 1], dst_b, ds_b)

        def body(i, carry):
            ka = 2 * i
            kb = 2 * i + 1
            pltpu.make_async_copy(g_hbm.at[src_v.at[ka]], rows_a, gs_a).wait()
            pltpu.make_async_copy(dst_hbm.at[wid * NCHUNK + ka], dst_a, ds_a).wait()
            pltpu.async_copy(rows_a, acc_sh.at[dst_a], ss_a, add=True)
            pltpu.make_async_copy(g_hbm.at[src_v.at[kb]], rows_b, gs_b).wait()
            pltpu.make_async_copy(dst_hbm.at[wid * NCHUNK + kb], dst_b, ds_b).wait()
            pltpu.async_copy(rows_b, acc_sh.at[dst_b], ss_b, add=True)
            pltpu.make_async_copy(rows_a, acc_sh.at[dst_a], ss_a).wait()
            pltpu.async_copy(g_hbm.at[src_v.at[ka + 2]], rows_a, gs_a)
            pltpu.async_copy(dst_hbm.at[wid * NCHUNK + ka + 2], dst_a, ds_a)
            pltpu.make_async_copy(rows_b, acc_sh.at[dst_b], ss_b).wait()

            @pl.when(kb + 2 < NCHUNK)
            def _():
                pltpu.async_copy(g_hbm.at[src_v.at[kb + 2]], rows_b, gs_b)
                pltpu.async_copy(dst_hbm.at[wid * NCHUNK + kb + 2], dst_b, ds_b)

            return carry

        lax.fori_loop(0, NCHUNK // 2, body, 0)
        last = NCHUNK - 1
        pltpu.make_async_copy(g_hbm.at[src_v.at[last]], rows_a, gs_a).wait()
        pltpu.make_async_copy(dst_hbm.at[wid * NCHUNK + last], dst_a, ds_a).wait()
        pltpu.async_copy(rows_a, acc_sh.at[dst_a], ss_a, add=True)
        pltpu.make_async_copy(rows_a, acc_sh.at[dst_a], ss_a).wait()
        plsc.subcore_barrier()
        pltpu.sync_copy(
            acc_sh.at[pl.ds(s * ROWS_PT, ROWS_PT)],
            out_hbm.at[c, pl.ds(s * ROWS_PT, ROWS_PT)],
        )

    return k(g, src3, dst3)


def _tc_prep(x_pad, W_pre, b_pre2, W_conv, deg_b):
    R = 1280
    grid = N_PAD // R

    def body(x_ref, wp_ref, bp_ref, wc_ref, degb_ref, ori_ref, g_ref):
        ori = (
            jnp.dot(x_ref[...], wp_ref[...], preferred_element_type=jnp.float32)
            + bp_ref[...]
        )
        ori_ref[...] = ori
        h2 = jnp.dot(ori, wc_ref[...], preferred_element_type=jnp.float32)
        g_ref[...] = lax.rsqrt(degb_ref[...]) * h2

    return pl.pallas_call(
        body,
        grid=(grid,),
        in_specs=[
            pl.BlockSpec((R, D), lambda i: (i, 0)),
            pl.BlockSpec((D, D), lambda i: (0, 0)),
            pl.BlockSpec((1, D), lambda i: (0, 0)),
            pl.BlockSpec((D, D), lambda i: (0, 0)),
            pl.BlockSpec((R, D), lambda i: (i, 0)),
        ],
        out_specs=[
            pl.BlockSpec((R, D), lambda i: (i, 0)),
            pl.BlockSpec((R, D), lambda i: (i, 0)),
        ],
        out_shape=[
            jax.ShapeDtypeStruct((N_PAD, D), jnp.float32),
            jax.ShapeDtypeStruct((N_PAD, D), jnp.float32),
        ],
    )(x_pad, W_pre, b_pre2, W_conv, deg_b)


def _tc_finish(P, g, deg_b, b_conv2):
    R = 1280
    grid = N_PAD // R

    def body(p_ref, g_ref, degb_ref, bc_ref, out_ref):
        tot = p_ref[0] + p_ref[1] + g_ref[...]
        out_ref[...] = jnp.maximum(
            lax.rsqrt(degb_ref[...]) * tot + bc_ref[...], 0.0
        )

    return pl.pallas_call(
        body,
        grid=(grid,),
        in_specs=[
            pl.BlockSpec((NC, R, D), lambda i: (0, i, 0)),
            pl.BlockSpec((R, D), lambda i: (i, 0)),
            pl.BlockSpec((R, D), lambda i: (i, 0)),
            pl.BlockSpec((1, D), lambda i: (0, 0)),
        ],
        out_specs=pl.BlockSpec((R, D), lambda i: (i, 0)),
        out_shape=jax.ShapeDtypeStruct((N_PAD, D), jnp.float32),
    )(P, g, deg_b, b_conv2)


def kernel(x, edge_index, W_pre, b_pre, W_conv, b_conv):
    n = x.shape[0]
    src3 = edge_index[0].reshape(NW, NCHUNK, CHUNK)
    dst3 = edge_index[1].reshape(NW, NCHUNK, CHUNK)
    x_pad = jnp.pad(x, ((0, N_PAD - n), (0, 0)))

    deg_parts = _sc_degree(dst3)
    deg = deg_parts[0] + deg_parts[1] + 1.0
    deg_b = jnp.broadcast_to(deg[:, None], (N_PAD, D))

    ori_pad, g_pad = _tc_prep(x_pad, W_pre, b_pre[None, :], W_conv, deg_b)
    P = _sc_scatter(g_pad, src3, dst3.reshape(NW * NCHUNK, CHUNK))
    h_pad = _tc_finish(P, g_pad, deg_b, b_conv[None, :])
    return h_pad[:n], ori_pad[:n]


# speedup vs baseline: 34.4276x; 1.0127x over previous
import functools

import jax
import jax.numpy as jnp
from jax import lax
from jax.experimental import pallas as pl
from jax.experimental.pallas import tpu as pltpu
from jax.experimental.pallas import tpu_sc as plsc

N = 10000
N_ACC = 10240
D = 128
NC, NS = 2, 16
NW = NC * NS
CHUNK = 80
NCHUNK = 125
ROWS_Z = N_ACC // NS
ROWS_OUT = N // NS
PACK_SHIFT = 14


def _sc_mesh():
    return plsc.VectorSubcoreMesh(
        core_axis_name="c", subcore_axis_name="s", num_cores=NC, num_subcores=NS
    )


def _sc_degree(dst3):
    @functools.partial(
        pl.kernel,
        mesh=_sc_mesh(),
        out_type=jax.ShapeDtypeStruct((NC, N_ACC), jnp.float32),
        scratch_types=[
            pltpu.VMEM((NCHUNK, CHUNK), jnp.int32),
            pltpu.VMEM((128,), jnp.float32),
            pltpu.VMEM((ROWS_Z,), jnp.float32),
            pltpu.VMEM_SHARED((N_ACC,), jnp.float32),
            pltpu.SemaphoreType.DMA,
        ],
    )
    def k(dst_hbm, out_hbm, dst_v, ones_v, zeros_v, acc_sh, sem):
        c = lax.axis_index("c")
        s = lax.axis_index("s")
        wid = c * NS + s
        pltpu.sync_copy(dst_hbm.at[wid], dst_v)
        for j in range(128 // 16):
            ones_v[pl.ds(j * 16, 16)] = jnp.full((16,), 1.0, jnp.float32)
        for j in range(ROWS_Z // 16):
            zeros_v[pl.ds(j * 16, 16)] = jnp.zeros((16,), jnp.float32)
        pltpu.sync_copy(zeros_v, acc_sh.at[pl.ds(s * ROWS_Z, ROWS_Z)])
        plsc.subcore_barrier()

        ones_c = ones_v.at[pl.ds(0, CHUNK)]

        def body(i, carry):
            for j in range(5):
                pltpu.async_copy(
                    ones_c, acc_sh.at[dst_v.at[i * 5 + j]], sem, add=True
                )
            for j in range(5):
                pltpu.make_async_copy(
                    ones_c, acc_sh.at[dst_v.at[i * 5 + j]], sem
                ).wait()
            return carry

        lax.fori_loop(0, NCHUNK // 5, body, 0)
        plsc.subcore_barrier()
        pltpu.sync_copy(
            acc_sh.at[pl.ds(s * ROWS_Z, ROWS_Z)],
            out_hbm.at[c, pl.ds(s * ROWS_Z, ROWS_Z)],
        )

    return k(dst3)


def _sc_scatter(g, packed3):
    @functools.partial(
        pl.kernel,
        mesh=_sc_mesh(),
        out_type=jax.ShapeDtypeStruct((NC, N_ACC, D), jnp.float32),
        scratch_types=[
            pltpu.VMEM((NCHUNK, CHUNK), jnp.int32),
            pltpu.VMEM((CHUNK,), jnp.int32),
            pltpu.VMEM((CHUNK,), jnp.int32),
            pltpu.VMEM((CHUNK,), jnp.int32),
            pltpu.VMEM((CHUNK,), jnp.int32),
            pltpu.VMEM((CHUNK, D), jnp.float32),
            pltpu.VMEM((CHUNK, D), jnp.float32),
            pltpu.VMEM_SHARED((N_ACC, D), jnp.float32),
            pltpu.SemaphoreType.DMA,
            pltpu.SemaphoreType.DMA,
            pltpu.SemaphoreType.DMA,
            pltpu.SemaphoreType.DMA,
        ],
    )
    def k(g_hbm, pk_hbm, out_hbm,
          pk_v, src_a, src_b, dst_a, dst_b, rows_a, rows_b, acc_sh,
          gs_a, gs_b, ss_a, ss_b):
        c = lax.axis_index("c")
        s = lax.axis_index("s")
        wid = c * NS + s
        pltpu.sync_copy(pk_hbm.at[wid], pk_v)

        def unpack(j, src_x, dst_x):
            for m in range(CHUNK // 16):
                v = pk_v[j, pl.ds(m * 16, 16)]
                src_x[pl.ds(m * 16, 16)] = lax.shift_right_logical(
                    v, jnp.full((16,), PACK_SHIFT, jnp.int32)
                )
                dst_x[pl.ds(m * 16, 16)] = lax.bitwise_and(
                    v, jnp.full((16,), (1 << PACK_SHIFT) - 1, jnp.int32)
                )

        def zrow(i, carry):
            for j in range(D // 16):
                rows_a[i, pl.ds(j * 16, 16)] = jnp.zeros((16,), jnp.float32)
            return carry

        lax.fori_loop(0, CHUNK, zrow, 0)
        for t in range(ROWS_Z // CHUNK):
            pltpu.sync_copy(rows_a, acc_sh.at[pl.ds(s * ROWS_Z + t * CHUNK, CHUNK)])
        plsc.subcore_barrier()

        unpack(0, src_a, dst_a)
        pltpu.async_copy(g_hbm.at[src_a], rows_a, gs_a)
        unpack(1, src_b, dst_b)
        pltpu.async_copy(g_hbm.at[src_b], rows_b, gs_b)

        def body(i, carry):
            ka = 2 * i
            kb = 2 * i + 1
            pltpu.make_async_copy(g_hbm.at[src_a], rows_a, gs_a).wait()
            pltpu.async_copy(rows_a, acc_sh.at[dst_a], ss_a, add=True)
            pltpu.make_async_copy(g_hbm.at[src_b], rows_b, gs_b).wait()
            pltpu.async_copy(rows_b, acc_sh.at[dst_b], ss_b, add=True)
            pltpu.make_async_copy(rows_a, acc_sh.at[dst_a], ss_a).wait()
            unpack(ka + 2, src_a, dst_a)
            pltpu.async_copy(g_hbm.at[src_a], rows_a, gs_a)
            pltpu.make_async_copy(rows_b, acc_sh.at[dst_b], ss_b).wait()

            @pl.when(kb + 2 < NCHUNK)
            def _():
                unpack(kb + 2, src_b, dst_b)
                pltpu.async_copy(g_hbm.at[src_b], rows_b, gs_b)

            return carry

        lax.fori_loop(0, NCHUNK // 2, body, 0)
        pltpu.make_async_copy(g_hbm.at[src_a], rows_a, gs_a).wait()
        pltpu.async_copy(rows_a, acc_sh.at[dst_a], ss_a, add=True)
        pltpu.make_async_copy(rows_a, acc_sh.at[dst_a], ss_a).wait()
        plsc.subcore_barrier()
        pltpu.sync_copy(
            acc_sh.at[pl.ds(s * ROWS_Z, ROWS_Z)],
            out_hbm.at[c, pl.ds(s * ROWS_Z, ROWS_Z)],
        )

    return k(g, packed3)


def _tc_prep(x, W_pre, b_pre2, W_conv, deg_b):
    R = 1000
    grid = N // R

    def body(x_ref, wp_ref, bp_ref, wc_ref, degb_ref, ori_ref, g_ref):
        ori = (
            jnp.dot(x_ref[...], wp_ref[...], preferred_element_type=jnp.float32)
            + bp_ref[...]
        )
        ori_ref[...] = ori
        h2 = jnp.dot(ori, wc_ref[...], preferred_element_type=jnp.float32)
        g_ref[...] = lax.rsqrt(degb_ref[...]) * h2

    return pl.pallas_call(
        body,
        grid=(grid,),
        in_specs=[
            pl.BlockSpec((R, D), lambda i: (i, 0)),
            pl.BlockSpec((D, D), lambda i: (0, 0)),
            pl.BlockSpec((1, D), lambda i: (0, 0)),
            pl.BlockSpec((D, D), lambda i: (0, 0)),
            pl.BlockSpec((R, D), lambda i: (i, 0)),
        ],
        out_specs=[
            pl.BlockSpec((R, D), lambda i: (i, 0)),
            pl.BlockSpec((R, D), lambda i: (i, 0)),
        ],
        out_shape=[
            jax.ShapeDtypeStruct((N, D), jnp.float32),
            jax.ShapeDtypeStruct((N, D), jnp.float32),
        ],
    )(x, W_pre, b_pre2, W_conv, deg_b)


def _tc_finish(P, g, deg_b, b_conv2):
    R = 1000
    grid = N // R

    def body(p_ref, g_ref, degb_ref, bc_ref, out_ref):
        tot = p_ref[0] + p_ref[1] + g_ref[...]
        out_ref[...] = jnp.maximum(
            lax.rsqrt(degb_ref[...]) * tot + bc_ref[...], 0.0
        )

    return pl.pallas_call(
        body,
        grid=(grid,),
        in_specs=[
            pl.BlockSpec((NC, R, D), lambda i: (0, i, 0)),
            pl.BlockSpec((R, D), lambda i: (i, 0)),
            pl.BlockSpec((R, D), lambda i: (i, 0)),
            pl.BlockSpec((1, D), lambda i: (0, 0)),
        ],
        out_specs=pl.BlockSpec((R, D), lambda i: (i, 0)),
        out_shape=jax.ShapeDtypeStruct((N, D), jnp.float32),
    )(P, g, deg_b, b_conv2)


def kernel(x, edge_index, W_pre, b_pre, W_conv, b_conv):
    src = edge_index[0]
    dst = edge_index[1]
    dst3 = dst.reshape(NW, NCHUNK, CHUNK)
    packed3 = ((src << PACK_SHIFT) | dst).reshape(NW, NCHUNK, CHUNK)

    deg_parts = _sc_degree(dst3)
    deg = (deg_parts[0] + deg_parts[1] + 1.0)[:N]
    deg_b = jnp.broadcast_to(deg[:, None], (N, D))

    ori, g = _tc_prep(x, W_pre, b_pre[None, :], W_conv, deg_b)
    P = _sc_scatter(g, packed3)
    h = _tc_finish(P, g, deg_b, b_conv[None, :])
    return h, ori


# speedup vs baseline: 40.5357x; 1.1774x over previous
import functools

import jax
import jax.numpy as jnp
from jax import lax
from jax.experimental import pallas as pl
from jax.experimental.pallas import tpu as pltpu
from jax.experimental.pallas import tpu_sc as plsc

N = 10000
N_ACC = 10240
D = 128
NC, NS = 2, 16
NW = NC * NS
CHUNK = 80
NCHUNK = 125
ROWS_Z = N_ACC // NS
ROWS_OUT = N // NS
PACK_SHIFT = 14


def _sc_mesh():
    return plsc.VectorSubcoreMesh(
        core_axis_name="c", subcore_axis_name="s", num_cores=NC, num_subcores=NS
    )


def _sc_degree(dst3):
    @functools.partial(
        pl.kernel,
        mesh=_sc_mesh(),
        out_type=jax.ShapeDtypeStruct((NC, N_ACC), jnp.float32),
        scratch_types=[
            pltpu.VMEM((NCHUNK, CHUNK), jnp.int32),
            pltpu.VMEM((128,), jnp.float32),
            pltpu.VMEM((ROWS_Z,), jnp.float32),
            pltpu.VMEM_SHARED((N_ACC,), jnp.float32),
            pltpu.SemaphoreType.DMA,
        ],
    )
    def k(dst_hbm, out_hbm, dst_v, ones_v, zeros_v, acc_sh, sem):
        c = lax.axis_index("c")
        s = lax.axis_index("s")
        wid = c * NS + s
        pltpu.sync_copy(dst_hbm.at[wid], dst_v)
        for j in range(128 // 16):
            ones_v[pl.ds(j * 16, 16)] = jnp.full((16,), 1.0, jnp.float32)
        for j in range(ROWS_Z // 16):
            zeros_v[pl.ds(j * 16, 16)] = jnp.zeros((16,), jnp.float32)
        pltpu.sync_copy(zeros_v, acc_sh.at[pl.ds(s * ROWS_Z, ROWS_Z)])
        plsc.subcore_barrier()

        ones_c = ones_v.at[pl.ds(0, CHUNK)]

        def body(i, carry):
            for j in range(5):
                pltpu.async_copy(
                    ones_c, acc_sh.at[dst_v.at[i * 5 + j]], sem, add=True
                )
            for j in range(5):
                pltpu.make_async_copy(
                    ones_c, acc_sh.at[dst_v.at[i * 5 + j]], sem
                ).wait()
            return carry

        lax.fori_loop(0, NCHUNK // 5, body, 0)
        plsc.subcore_barrier()
        pltpu.sync_copy(
            acc_sh.at[pl.ds(s * ROWS_Z, ROWS_Z)],
            out_hbm.at[c, pl.ds(s * ROWS_Z, ROWS_Z)],
        )

    return k(dst3)


def _sc_scatter(g, packed3):
    @functools.partial(
        pl.kernel,
        mesh=_sc_mesh(),
        out_type=jax.ShapeDtypeStruct((NC, N_ACC, D), jnp.float32),
        scratch_types=[
            [pltpu.VMEM((CHUNK,), jnp.int32)] * 3,
            [pltpu.VMEM((CHUNK,), jnp.int32)] * 3,
            [pltpu.VMEM((CHUNK,), jnp.int32)] * 3,
            [pltpu.VMEM((CHUNK, D), jnp.float32)] * 3,
            pltpu.VMEM_SHARED((N_ACC, D), jnp.float32),
            [pltpu.SemaphoreType.DMA] * 3,
            [pltpu.SemaphoreType.DMA] * 3,
            [pltpu.SemaphoreType.DMA] * 3,
        ],
    )
    def k(g_hbm, pk_hbm, out_hbm,
          pkb, srcb, dstb, rows, acc_sh, ps, gs, ss):
        c = lax.axis_index("c")
        s = lax.axis_index("s")
        wid = c * NS + s
        base = wid * NCHUNK

        def unpack(pk_x, src_x, dst_x):
            for m in range(CHUNK // 16):
                v = pk_x[pl.ds(m * 16, 16)]
                src_x[pl.ds(m * 16, 16)] = lax.shift_right_logical(
                    v, jnp.full((16,), PACK_SHIFT, jnp.int32)
                )
                dst_x[pl.ds(m * 16, 16)] = lax.bitwise_and(
                    v, jnp.full((16,), (1 << PACK_SHIFT) - 1, jnp.int32)
                )

        def zrow(i, carry):
            for j in range(D // 16):
                rows[0][i, pl.ds(j * 16, 16)] = jnp.zeros((16,), jnp.float32)
            return carry

        lax.fori_loop(0, CHUNK, zrow, 0)
        for t in range(ROWS_Z // CHUNK):
            pltpu.sync_copy(rows[0], acc_sh.at[pl.ds(s * ROWS_Z + t * CHUNK, CHUNK)])
        plsc.subcore_barrier()

        for m in range(3):
            pltpu.async_copy(pk_hbm.at[base + m], pkb[m], ps[m])
        for m in range(3):
            pltpu.make_async_copy(pk_hbm.at[base + m], pkb[m], ps[m]).wait()
            unpack(pkb[m], srcb[m], dstb[m])
            pltpu.async_copy(g_hbm.at[srcb[m]], rows[m], gs[m])
            pltpu.async_copy(pk_hbm.at[base + m + 3], pkb[m], ps[m])

        def body(i, carry):
            k0 = 3 * i
            for m in range(3):
                pltpu.make_async_copy(g_hbm.at[srcb[m]], rows[m], gs[m]).wait()
                pltpu.async_copy(rows[m], acc_sh.at[dstb[m]], ss[m], add=True)
            for m in range(3):
                k = k0 + m
                pltpu.make_async_copy(rows[m], acc_sh.at[dstb[m]], ss[m]).wait()

                @pl.when(k + 3 < NCHUNK)
                def _():
                    pltpu.make_async_copy(
                        pk_hbm.at[base + k + 3], pkb[m], ps[m]
                    ).wait()
                    unpack(pkb[m], srcb[m], dstb[m])
                    pltpu.async_copy(g_hbm.at[srcb[m]], rows[m], gs[m])

                    @pl.when(k + 6 < NCHUNK)
                    def _():
                        pltpu.async_copy(pk_hbm.at[base + k + 6], pkb[m], ps[m])

            return carry

        lax.fori_loop(0, NCHUNK // 3, body, 0)
        for m in range(NCHUNK - 3 * (NCHUNK // 3)):
            pltpu.make_async_copy(g_hbm.at[srcb[m]], rows[m], gs[m]).wait()
            pltpu.async_copy(rows[m], acc_sh.at[dstb[m]], ss[m], add=True)
            pltpu.make_async_copy(rows[m], acc_sh.at[dstb[m]], ss[m]).wait()
        plsc.subcore_barrier()
        pltpu.sync_copy(
            acc_sh.at[pl.ds(s * ROWS_Z, ROWS_Z)],
            out_hbm.at[c, pl.ds(s * ROWS_Z, ROWS_Z)],
        )

    return k(g, packed3)


def _tc_prep(x, W_pre, b_pre2, W_conv, deg_b):
    R = 1000
    grid = N // R

    def body(x_ref, wp_ref, bp_ref, wc_ref, degb_ref, ori_ref, g_ref):
        ori = (
            jnp.dot(x_ref[...], wp_ref[...], preferred_element_type=jnp.float32)
            + bp_ref[...]
        )
        ori_ref[...] = ori
        h2 = jnp.dot(ori, wc_ref[...], preferred_element_type=jnp.float32)
        g_ref[...] = lax.rsqrt(degb_ref[...]) * h2

    return pl.pallas_call(
        body,
        grid=(grid,),
        in_specs=[
            pl.BlockSpec((R, D), lambda i: (i, 0)),
            pl.BlockSpec((D, D), lambda i: (0, 0)),
            pl.BlockSpec((1, D), lambda i: (0, 0)),
            pl.BlockSpec((D, D), lambda i: (0, 0)),
            pl.BlockSpec((R, D), lambda i: (i, 0)),
        ],
        out_specs=[
            pl.BlockSpec((R, D), lambda i: (i, 0)),
            pl.BlockSpec((R, D), lambda i: (i, 0)),
        ],
        out_shape=[
            jax.ShapeDtypeStruct((N, D), jnp.float32),
            jax.ShapeDtypeStruct((N, D), jnp.float32),
        ],
    )(x, W_pre, b_pre2, W_conv, deg_b)


def _tc_finish(P, g, deg_b, b_conv2):
    R = 1000
    grid = N // R

    def body(p_ref, g_ref, degb_ref, bc_ref, out_ref):
        tot = p_ref[0] + p_ref[1] + g_ref[...]
        out_ref[...] = jnp.maximum(
            lax.rsqrt(degb_ref[...]) * tot + bc_ref[...], 0.0
        )

    return pl.pallas_call(
        body,
        grid=(grid,),
        in_specs=[
            pl.BlockSpec((NC, R, D), lambda i: (0, i, 0)),
            pl.BlockSpec((R, D), lambda i: (i, 0)),
            pl.BlockSpec((R, D), lambda i: (i, 0)),
            pl.BlockSpec((1, D), lambda i: (0, 0)),
        ],
        out_specs=pl.BlockSpec((R, D), lambda i: (i, 0)),
        out_shape=jax.ShapeDtypeStruct((N, D), jnp.float32),
    )(P, g, deg_b, b_conv2)


def kernel(x, edge_index, W_pre, b_pre, W_conv, b_conv):
    src = edge_index[0]
    dst = edge_index[1]
    dst3 = dst.reshape(NW, NCHUNK, CHUNK)
    packed3 = ((src << PACK_SHIFT) | dst).reshape(NW * NCHUNK, CHUNK)

    deg_parts = _sc_degree(dst3)
    deg = (deg_parts[0] + deg_parts[1] + 1.0)[:N]
    deg_b = jnp.broadcast_to(deg[:, None], (N, D))

    ori, g = _tc_prep(x, W_pre, b_pre[None, :], W_conv, deg_b)
    P = _sc_scatter(g, packed3)
    h = _tc_finish(P, g, deg_b, b_conv[None, :])
    return h, ori
